# initial kernel scaffold (unmeasured)
import jax
import jax.numpy as jnp
from jax import lax
from jax.experimental import pallas as pl
from jax.experimental.pallas import tpu as pltpu

N_DEV = 4
SQ_PER = 512
D_MODEL = 1024
N_HEADS = 8
D_HEAD = 128
SKV = 2048
SCALE = 0.08838834764831843


def kernel(x, Wq, Wo, K_ext, V_ext):
    i = lax.axis_index("i")

    x2 = x[0].astype(jnp.bfloat16)
    wq = Wq.astype(jnp.bfloat16)
    wo = Wo.astype(jnp.bfloat16)
    k_loc = lax.dynamic_slice_in_dim(K_ext[0], i * N_HEADS, N_HEADS, axis=1)
    v_loc = lax.dynamic_slice_in_dim(V_ext[0], i * N_HEADS, N_HEADS, axis=1)
    k_loc = jnp.transpose(k_loc, (1, 0, 2)).astype(jnp.bfloat16)
    v_loc = jnp.transpose(v_loc, (1, 0, 2)).astype(jnp.bfloat16)

    def body(x_ref, wq_ref, wo_ref, k_ref, v_ref, out_ref,
             xg_ref, rs_ref, rcv_ref, o_ref,
             ag_send_sems, ag_recv_sems, rs_send_sems, rs_recv_sems):
        my = lax.axis_index("i")
        left = lax.rem(my + N_DEV - 1, N_DEV)
        right = lax.rem(my + 1, N_DEV)

        barrier_sem = pltpu.get_barrier_semaphore()
        for nbr in (left, right):
            pl.semaphore_signal(
                barrier_sem, inc=1,
                device_id=(nbr,), device_id_type=pl.DeviceIdType.MESH,
            )
        pl.semaphore_wait(barrier_sem, 2)

        pl.store(xg_ref, (pl.ds(my, 1), slice(None), slice(None)),
                 x_ref[...][None])
        for h in range(N_DEV - 1):
            send_chunk = lax.rem(my - h + N_DEV, N_DEV)
            rdma = pltpu.make_async_remote_copy(
                src_ref=xg_ref.at[send_chunk],
                dst_ref=xg_ref.at[send_chunk],
                send_sem=ag_send_sems.at[h],
                recv_sem=ag_recv_sems.at[h],
                device_id=(right,),
                device_id_type=pl.DeviceIdType.MESH,
            )
            rdma.start()
            rdma.wait()

        for c in range(N_DEV):
            xc = xg_ref[c]
            for h in range(N_HEADS):
                wq_h = wq_ref[:, h * D_HEAD:(h + 1) * D_HEAD]
                q = jax.lax.dot_general(
                    xc, wq_h, (((1,), (0,)), ((), ())),
                    preferred_element_type=jnp.float32,
                ).astype(jnp.bfloat16)
                s = jax.lax.dot_general(
                    q, k_ref[h], (((1,), (1,)), ((), ())),
                    preferred_element_type=jnp.float32,
                ) * SCALE
                m = jnp.max(s, axis=1, keepdims=True)
                p = jnp.exp(s - m)
                l = jnp.sum(p, axis=1, keepdims=True)
                o = jax.lax.dot_general(
                    p.astype(jnp.bfloat16), v_ref[h], (((1,), (0,)), ((), ())),
                    preferred_element_type=jnp.float32,
                ) / l
                o_ref[:, h * D_HEAD:(h + 1) * D_HEAD] = o.astype(jnp.bfloat16)
            part = jax.lax.dot_general(
                o_ref[...], wo_ref[...], (((1,), (0,)), ((), ())),
                preferred_element_type=jnp.float32,
            )
            rs_ref[c] = part.astype(jnp.bfloat16)

        for t in range(N_DEV - 1):
            c_send = lax.rem(my - 1 - t + 2 * N_DEV, N_DEV)
            rdma = pltpu.make_async_remote_copy(
                src_ref=rs_ref.at[c_send],
                dst_ref=rcv_ref.at[t],
                send_sem=rs_send_sems.at[t],
                recv_sem=rs_recv_sems.at[t],
                device_id=(right,),
                device_id_type=pl.DeviceIdType.MESH,
            )
            rdma.start()
            rdma.wait()
            c_recv = lax.rem(my - 2 - t + 2 * N_DEV, N_DEV)
            acc = pl.load(rs_ref, (pl.ds(c_recv, 1), slice(None), slice(None)))
            pl.store(rs_ref, (pl.ds(c_recv, 1), slice(None), slice(None)),
                     acc + rcv_ref[t][None])

        out_ref[...] = pl.load(
            rs_ref, (pl.ds(my, 1), slice(None), slice(None))
        )[0].astype(jnp.float32)

    out = pl.pallas_call(
        body,
        out_shape=jax.ShapeDtypeStruct((SQ_PER, D_MODEL), jnp.float32),
        in_specs=[pl.BlockSpec(memory_space=pltpu.VMEM)] * 5,
        out_specs=pl.BlockSpec(memory_space=pltpu.VMEM),
        scratch_shapes=[
            pltpu.VMEM((N_DEV, SQ_PER, D_MODEL), jnp.bfloat16),
            pltpu.VMEM((N_DEV, SQ_PER, D_MODEL), jnp.bfloat16),
            pltpu.VMEM((N_DEV - 1, SQ_PER, D_MODEL), jnp.bfloat16),
            pltpu.VMEM((SQ_PER, N_HEADS * D_HEAD), jnp.bfloat16),
            pltpu.SemaphoreType.DMA((N_DEV - 1,)),
            pltpu.SemaphoreType.DMA((N_DEV - 1,)),
            pltpu.SemaphoreType.DMA((N_DEV - 1,)),
            pltpu.SemaphoreType.DMA((N_DEV - 1,)),
        ],
        compiler_params=pltpu.CompilerParams(collective_id=0),
    )(x2, wq, wo, k_loc, v_loc)

    return out[None]


# baseline (device time: 181345 ns/iter reference)
import jax
import jax.numpy as jnp
from jax import lax
from jax.experimental import pallas as pl
from jax.experimental.pallas import tpu as pltpu

N_DEV = 4
SQ_PER = 512
D_MODEL = 1024
N_HEADS = 8
D_HEAD = 128
SKV = 2048
SCALE = 0.08838834764831843


def kernel(x, Wq, Wo, K_ext, V_ext):
    i = lax.axis_index("i")

    x2 = x[0].astype(jnp.bfloat16)
    wq = Wq.astype(jnp.bfloat16)
    wo = Wo.astype(jnp.bfloat16)
    k_loc = lax.dynamic_slice_in_dim(K_ext[0], i * N_HEADS, N_HEADS, axis=1)
    v_loc = lax.dynamic_slice_in_dim(V_ext[0], i * N_HEADS, N_HEADS, axis=1)
    k_loc = jnp.transpose(k_loc, (1, 0, 2)).astype(jnp.bfloat16)
    v_loc = jnp.transpose(v_loc, (1, 0, 2)).astype(jnp.bfloat16)

    def body(x_ref, wq_ref, wo_ref, k_ref, v_ref, out_ref,
             xg_ref, rs_ref, rcv_ref, o_ref,
             ag_send_sems, ag_recv_sems, rs_send_sems, rs_recv_sems):
        my = lax.axis_index("i")
        left = lax.rem(my + N_DEV - 1, N_DEV)
        right = lax.rem(my + 1, N_DEV)

        barrier_sem = pltpu.get_barrier_semaphore()
        for nbr in (left, right):
            pl.semaphore_signal(
                barrier_sem, inc=1,
                device_id=(nbr,), device_id_type=pl.DeviceIdType.MESH,
            )
        pl.semaphore_wait(barrier_sem, 2)

        xg_ref[pl.ds(my, 1)] = x_ref[...][None]
        for h in range(N_DEV - 1):
            send_chunk = lax.rem(my - h + N_DEV, N_DEV)
            rdma = pltpu.make_async_remote_copy(
                src_ref=xg_ref.at[send_chunk],
                dst_ref=xg_ref.at[send_chunk],
                send_sem=ag_send_sems.at[h],
                recv_sem=ag_recv_sems.at[h],
                device_id=(right,),
                device_id_type=pl.DeviceIdType.MESH,
            )
            rdma.start()
            rdma.wait()

        for c in range(N_DEV):
            xc = xg_ref[c]
            for h in range(N_HEADS):
                wq_h = wq_ref[:, h * D_HEAD:(h + 1) * D_HEAD]
                q = jax.lax.dot_general(
                    xc, wq_h, (((1,), (0,)), ((), ())),
                    preferred_element_type=jnp.float32,
                ).astype(jnp.bfloat16)
                s = jax.lax.dot_general(
                    q, k_ref[h], (((1,), (1,)), ((), ())),
                    preferred_element_type=jnp.float32,
                ) * SCALE
                m = jnp.max(s, axis=1, keepdims=True)
                p = jnp.exp(s - m)
                l = jnp.sum(p, axis=1, keepdims=True)
                o = jax.lax.dot_general(
                    p.astype(jnp.bfloat16), v_ref[h], (((1,), (0,)), ((), ())),
                    preferred_element_type=jnp.float32,
                ) / l
                o_ref[:, h * D_HEAD:(h + 1) * D_HEAD] = o.astype(jnp.bfloat16)
            part = jax.lax.dot_general(
                o_ref[...], wo_ref[...], (((1,), (0,)), ((), ())),
                preferred_element_type=jnp.float32,
            )
            rs_ref[c] = part.astype(jnp.bfloat16)

        for t in range(N_DEV - 1):
            c_send = lax.rem(my - 1 - t + 2 * N_DEV, N_DEV)
            rdma = pltpu.make_async_remote_copy(
                src_ref=rs_ref.at[c_send],
                dst_ref=rcv_ref.at[t],
                send_sem=rs_send_sems.at[t],
                recv_sem=rs_recv_sems.at[t],
                device_id=(right,),
                device_id_type=pl.DeviceIdType.MESH,
            )
            rdma.start()
            rdma.wait()
            c_recv = lax.rem(my - 2 - t + 2 * N_DEV, N_DEV)
            acc = rs_ref[pl.ds(c_recv, 1)]
            rs_ref[pl.ds(c_recv, 1)] = acc + rcv_ref[t][None]

        out_ref[...] = rs_ref[pl.ds(my, 1)][0].astype(jnp.float32)

    out = pl.pallas_call(
        body,
        out_shape=jax.ShapeDtypeStruct((SQ_PER, D_MODEL), jnp.float32),
        in_specs=[pl.BlockSpec(memory_space=pltpu.VMEM)] * 5,
        out_specs=pl.BlockSpec(memory_space=pltpu.VMEM),
        scratch_shapes=[
            pltpu.VMEM((N_DEV, SQ_PER, D_MODEL), jnp.bfloat16),
            pltpu.VMEM((N_DEV, SQ_PER, D_MODEL), jnp.bfloat16),
            pltpu.VMEM((N_DEV - 1, SQ_PER, D_MODEL), jnp.bfloat16),
            pltpu.VMEM((SQ_PER, N_HEADS * D_HEAD), jnp.bfloat16),
            pltpu.SemaphoreType.DMA((N_DEV - 1,)),
            pltpu.SemaphoreType.DMA((N_DEV - 1,)),
            pltpu.SemaphoreType.DMA((N_DEV - 1,)),
            pltpu.SemaphoreType.DMA((N_DEV - 1,)),
        ],
        compiler_params=pltpu.CompilerParams(
            collective_id=0, vmem_limit_bytes=64 * 1024 * 1024,
        ),
    )(x2, wq, wo, k_loc, v_loc)

    return out[None]


# device time: 125733 ns/iter; 1.4423x vs baseline; 1.4423x over previous
import jax
import jax.numpy as jnp
from jax import lax
from jax.experimental import pallas as pl
from jax.experimental.pallas import tpu as pltpu

N_DEV = 4
SQ_PER = 512
D_MODEL = 1024
N_HEADS = 8
D_HEAD = 128
SKV = 2048
SCALE = 0.08838834764831843


def kernel(x, Wq, Wo, K_ext, V_ext):
    i = lax.axis_index("i")

    x2 = x[0].astype(jnp.bfloat16)
    wq = Wq.astype(jnp.bfloat16)
    wo = Wo.astype(jnp.bfloat16)
    k_loc = lax.dynamic_slice_in_dim(K_ext[0], i * N_HEADS, N_HEADS, axis=1)
    v_loc = lax.dynamic_slice_in_dim(V_ext[0], i * N_HEADS, N_HEADS, axis=1)
    k_loc = jnp.transpose(k_loc, (1, 0, 2)).astype(jnp.bfloat16)
    v_loc = jnp.transpose(v_loc, (1, 0, 2)).astype(jnp.bfloat16)

    def body(x_ref, wq_ref, wo_ref, k_ref, v_ref, out_ref,
             xg_ref, rs_ref, rcv_ref, o_ref,
             ag_send_sems, ag_recv_sems, rs_send_sems, rs_recv_sems):
        my = lax.axis_index("i")
        left = lax.rem(my + N_DEV - 1, N_DEV)
        right = lax.rem(my + 1, N_DEV)

        barrier_sem = pltpu.get_barrier_semaphore()
        for nbr in (left, right):
            pl.semaphore_signal(
                barrier_sem, inc=1,
                device_id=(nbr,), device_id_type=pl.DeviceIdType.MESH,
            )
        pl.semaphore_wait(barrier_sem, 2)

        def ag_rdma(h):
            send_chunk = lax.rem(my - h + N_DEV, N_DEV)
            return pltpu.make_async_remote_copy(
                src_ref=xg_ref.at[send_chunk],
                dst_ref=xg_ref.at[send_chunk],
                send_sem=ag_send_sems.at[h],
                recv_sem=ag_recv_sems.at[h],
                device_id=(right,),
                device_id_type=pl.DeviceIdType.MESH,
            )

        def rs_rdma(t):
            c_send = lax.rem(my - 1 - t + 2 * N_DEV, N_DEV)
            return pltpu.make_async_remote_copy(
                src_ref=rs_ref.at[c_send],
                dst_ref=rcv_ref.at[t],
                send_sem=rs_send_sems.at[t],
                recv_sem=rs_recv_sems.at[t],
                device_id=(right,),
                device_id_type=pl.DeviceIdType.MESH,
            )

        def rs_accum(t):
            c_recv = lax.rem(my - 2 - t + 2 * N_DEV, N_DEV)
            acc = rs_ref[pl.ds(c_recv, 1)]
            rs_ref[pl.ds(c_recv, 1)] = acc + rcv_ref[t][None]

        def compute_chunk(c):
            xc = xg_ref[pl.ds(c, 1)][0]
            for h in range(N_HEADS):
                wq_h = wq_ref[:, h * D_HEAD:(h + 1) * D_HEAD]
                q = jax.lax.dot_general(
                    xc, wq_h, (((1,), (0,)), ((), ())),
                    preferred_element_type=jnp.float32,
                ).astype(jnp.bfloat16)
                s = jax.lax.dot_general(
                    q, k_ref[h], (((1,), (1,)), ((), ())),
                    preferred_element_type=jnp.float32,
                ) * SCALE
                m = jnp.max(s, axis=1, keepdims=True)
                p = jnp.exp(s - m)
                l = jnp.sum(p, axis=1, keepdims=True)
                o = jax.lax.dot_general(
                    p.astype(jnp.bfloat16), v_ref[h], (((1,), (0,)), ((), ())),
                    preferred_element_type=jnp.float32,
                ) / l
                o_ref[:, h * D_HEAD:(h + 1) * D_HEAD] = o.astype(jnp.bfloat16)
            part = jax.lax.dot_general(
                o_ref[...], wo_ref[...], (((1,), (0,)), ((), ())),
                preferred_element_type=jnp.float32,
            )
            rs_ref[pl.ds(c, 1)] = part.astype(jnp.bfloat16)[None]

        xg_ref[pl.ds(my, 1)] = x_ref[...][None]
        ag0, ag1, ag2 = ag_rdma(0), ag_rdma(1), ag_rdma(2)
        rs0, rs1, rs2 = rs_rdma(0), rs_rdma(1), rs_rdma(2)

        ag0.start()
        compute_chunk(my)
        ag0.wait()
        ag1.start()
        compute_chunk(lax.rem(my + 3, N_DEV))
        rs0.start()
        ag1.wait()
        ag2.start()
        compute_chunk(lax.rem(my + 2, N_DEV))
        rs0.wait()
        rs_accum(0)
        rs1.start()
        ag2.wait()
        compute_chunk(lax.rem(my + 1, N_DEV))
        rs1.wait()
        rs_accum(1)
        rs2.start()
        rs2.wait()
        rs_accum(2)

        out_ref[...] = rs_ref[pl.ds(my, 1)][0].astype(jnp.float32)

    out = pl.pallas_call(
        body,
        out_shape=jax.ShapeDtypeStruct((SQ_PER, D_MODEL), jnp.float32),
        in_specs=[pl.BlockSpec(memory_space=pltpu.VMEM)] * 5,
        out_specs=pl.BlockSpec(memory_space=pltpu.VMEM),
        scratch_shapes=[
            pltpu.VMEM((N_DEV, SQ_PER, D_MODEL), jnp.bfloat16),
            pltpu.VMEM((N_DEV, SQ_PER, D_MODEL), jnp.bfloat16),
            pltpu.VMEM((N_DEV - 1, SQ_PER, D_MODEL), jnp.bfloat16),
            pltpu.VMEM((SQ_PER, N_HEADS * D_HEAD), jnp.bfloat16),
            pltpu.SemaphoreType.DMA((N_DEV - 1,)),
            pltpu.SemaphoreType.DMA((N_DEV - 1,)),
            pltpu.SemaphoreType.DMA((N_DEV - 1,)),
            pltpu.SemaphoreType.DMA((N_DEV - 1,)),
        ],
        compiler_params=pltpu.CompilerParams(
            collective_id=0, vmem_limit_bytes=64 * 1024 * 1024,
        ),
    )(x2, wq, wo, k_loc, v_loc)

    return out[None]


# device time: 115708 ns/iter; 1.5673x vs baseline; 1.0866x over previous
import jax
import jax.numpy as jnp
from jax import lax
from jax.experimental import pallas as pl
from jax.experimental.pallas import tpu as pltpu

N_DEV = 4
SQ_PER = 512
D_MODEL = 1024
N_HEADS = 8
D_HEAD = 128
SKV = 2048
SCALE = 0.08838834764831843


def kernel(x, Wq, Wo, K_ext, V_ext):
    i = lax.axis_index("i")

    x2 = x[0].astype(jnp.bfloat16)
    wq = (Wq * SCALE).astype(jnp.bfloat16)
    wo = Wo.astype(jnp.bfloat16)
    k_loc = lax.dynamic_slice_in_dim(K_ext[0], i * N_HEADS, N_HEADS, axis=1)
    v_loc = lax.dynamic_slice_in_dim(V_ext[0], i * N_HEADS, N_HEADS, axis=1)
    k_loc = jnp.transpose(k_loc, (1, 0, 2)).astype(jnp.bfloat16)
    v_loc = jnp.transpose(v_loc, (1, 0, 2)).astype(jnp.bfloat16)

    def body(x_ref, wq_ref, wo_ref, k_ref, v_ref, out_ref,
             xg_ref, rs_ref, rcv_ref, o_ref,
             ag_send_sems, ag_recv_sems, rs_send_sems, rs_recv_sems):
        my = lax.axis_index("i")
        left = lax.rem(my + N_DEV - 1, N_DEV)
        right = lax.rem(my + 1, N_DEV)

        barrier_sem = pltpu.get_barrier_semaphore()
        for nbr in (left, right):
            pl.semaphore_signal(
                barrier_sem, inc=1,
                device_id=(nbr,), device_id_type=pl.DeviceIdType.MESH,
            )
        pl.semaphore_wait(barrier_sem, 2)

        def ag_rdma(h):
            send_chunk = lax.rem(my - h + N_DEV, N_DEV)
            return pltpu.make_async_remote_copy(
                src_ref=xg_ref.at[send_chunk],
                dst_ref=xg_ref.at[send_chunk],
                send_sem=ag_send_sems.at[h],
                recv_sem=ag_recv_sems.at[h],
                device_id=(right,),
                device_id_type=pl.DeviceIdType.MESH,
            )

        def rs_rdma(t):
            c_send = lax.rem(my - 1 - t + 2 * N_DEV, N_DEV)
            return pltpu.make_async_remote_copy(
                src_ref=rs_ref.at[c_send],
                dst_ref=rcv_ref.at[t],
                send_sem=rs_send_sems.at[t],
                recv_sem=rs_recv_sems.at[t],
                device_id=(right,),
                device_id_type=pl.DeviceIdType.MESH,
            )

        def rs_accum(t):
            c_recv = lax.rem(my - 2 - t + 2 * N_DEV, N_DEV)
            acc = rs_ref[pl.ds(c_recv, 1)]
            rs_ref[pl.ds(c_recv, 1)] = acc + rcv_ref[t][None]

        def compute_chunk(c):
            xc = xg_ref[pl.ds(c, 1)][0]
            for h in range(N_HEADS):
                wq_h = wq_ref[:, h * D_HEAD:(h + 1) * D_HEAD]
                q = jax.lax.dot_general(
                    xc, wq_h, (((1,), (0,)), ((), ())),
                    preferred_element_type=jnp.float32,
                ).astype(jnp.bfloat16)
                s = jax.lax.dot_general(
                    q, k_ref[h], (((1,), (1,)), ((), ())),
                    preferred_element_type=jnp.float32,
                )
                p = jnp.exp(s)
                l = jnp.sum(p, axis=1, keepdims=True)
                o = jax.lax.dot_general(
                    p.astype(jnp.bfloat16), v_ref[h], (((1,), (0,)), ((), ())),
                    preferred_element_type=jnp.float32,
                ) / l
                o_ref[:, h * D_HEAD:(h + 1) * D_HEAD] = o.astype(jnp.bfloat16)
            part = jax.lax.dot_general(
                o_ref[...], wo_ref[...], (((1,), (0,)), ((), ())),
                preferred_element_type=jnp.float32,
            )
            rs_ref[pl.ds(c, 1)] = part.astype(jnp.bfloat16)[None]

        xg_ref[pl.ds(my, 1)] = x_ref[...][None]
        ag0, ag1, ag2 = ag_rdma(0), ag_rdma(1), ag_rdma(2)
        rs0, rs1, rs2 = rs_rdma(0), rs_rdma(1), rs_rdma(2)

        ag0.start()
        compute_chunk(my)
        ag0.wait()
        ag1.start()
        compute_chunk(lax.rem(my + 3, N_DEV))
        rs0.start()
        ag1.wait()
        ag2.start()
        compute_chunk(lax.rem(my + 2, N_DEV))
        rs0.wait()
        rs_accum(0)
        rs1.start()
        ag2.wait()
        compute_chunk(lax.rem(my + 1, N_DEV))
        rs1.wait()
        rs_accum(1)
        rs2.start()
        rs2.wait()
        rs_accum(2)

        out_ref[...] = rs_ref[pl.ds(my, 1)][0].astype(jnp.float32)

    out = pl.pallas_call(
        body,
        out_shape=jax.ShapeDtypeStruct((SQ_PER, D_MODEL), jnp.float32),
        in_specs=[pl.BlockSpec(memory_space=pltpu.VMEM)] * 5,
        out_specs=pl.BlockSpec(memory_space=pltpu.VMEM),
        scratch_shapes=[
            pltpu.VMEM((N_DEV, SQ_PER, D_MODEL), jnp.bfloat16),
            pltpu.VMEM((N_DEV, SQ_PER, D_MODEL), jnp.bfloat16),
            pltpu.VMEM((N_DEV - 1, SQ_PER, D_MODEL), jnp.bfloat16),
            pltpu.VMEM((SQ_PER, N_HEADS * D_HEAD), jnp.bfloat16),
            pltpu.SemaphoreType.DMA((N_DEV - 1,)),
            pltpu.SemaphoreType.DMA((N_DEV - 1,)),
            pltpu.SemaphoreType.DMA((N_DEV - 1,)),
            pltpu.SemaphoreType.DMA((N_DEV - 1,)),
        ],
        compiler_params=pltpu.CompilerParams(
            collective_id=0, vmem_limit_bytes=64 * 1024 * 1024,
        ),
    )(x2, wq, wo, k_loc, v_loc)

    return out[None]
